# Initial kernel scaffold; baseline (speedup 1.0000x reference)
import jax
import jax.numpy as jnp
from jax.experimental import pallas as pl


def kernel(x, gate_w, gate_b, W1, b1, W2, b2):
    raise NotImplementedError("write your pallas kernel here")


# speedup vs baseline: 4.3847x; 4.3847x over previous
import functools

import jax
import jax.numpy as jnp
from jax import lax
from jax.experimental import pallas as pl
from jax.experimental.pallas import tpu as pltpu
from jax.experimental.pallas import tpu_sc as plsc

D = 1024
E = 8
H = 2048
T = 2048
M = 256
NT = T // M + (E - 1)
P = NT * M
NTP = 128


def _route_body(x_ref, gw_ref, gb_ref, pos_ref, te_ref):
    xs = x_ref[...]
    gw = gw_ref[...]
    scoresT = lax.dot_general(gw, xs, (((0,), (1,)), ((), ())),
                              preferred_element_type=jnp.float32)
    scoresT = scoresT + gb_ref[...].reshape(E, 1)

    eids = lax.broadcasted_iota(jnp.int32, (E, T), 0)
    mx = jnp.max(scoresT, axis=0, keepdims=True)
    selT = jnp.min(jnp.where(scoresT == mx, eids, E), axis=0, keepdims=True)
    oh = (eids == selT).astype(jnp.int32)

    c = oh
    k = 1
    while k < T:
        c = c + jnp.concatenate(
            [jnp.zeros((E, k), jnp.int32), c[:, : T - k]], axis=1)
        k *= 2

    counts = c[:, T - 1 : T]
    caps = jnp.bitwise_and(counts + (M - 1), -M)
    ic = caps
    k = 1
    while k < E:
        ic = ic + jnp.concatenate(
            [jnp.zeros((k, 1), jnp.int32), ic[: E - k, :]], axis=0)
        k *= 2
    offs = ic - caps

    rank = jnp.sum(c * oh, axis=0, keepdims=True) - 1
    base = jnp.sum(oh * offs, axis=0, keepdims=True)
    pos_ref[...] = base + rank

    total = ic[E - 1 : E, :]
    tstart = lax.broadcasted_iota(jnp.int32, (1, NTP), 1) * M
    tstart = jnp.minimum(tstart, total - M)
    te_ref[...] = jnp.sum((tstart >= ic).astype(jnp.int32), axis=0,
                          keepdims=True)


def _route(x2, gate_w, gate_b2):
    return pl.pallas_call(
        _route_body,
        out_shape=(
            jax.ShapeDtypeStruct((1, T), jnp.int32),
            jax.ShapeDtypeStruct((1, NTP), jnp.int32),
        ),
    )(x2, gate_w, gate_b2)


_NC = 2
_NS = 16
_NW = _NC * _NS
ROWS_W = T // _NW


@functools.lru_cache(maxsize=None)
def _sc_kernels():
    mesh = plsc.VectorSubcoreMesh(
        core_axis_name="c", subcore_axis_name="s",
        num_cores=_NC, num_subcores=_NS)
    scratch = [
        pltpu.VMEM((ROWS_W,), jnp.int32),
        pltpu.VMEM((ROWS_W, D), jnp.float32),
        pltpu.SemaphoreType.DMA,
    ]

    @functools.partial(
        pl.kernel,
        out_type=jax.ShapeDtypeStruct((P, D), jnp.float32),
        mesh=mesh,
        scratch_types=scratch,
    )
    def dispatch_k(x_hbm, pos_hbm, xs_hbm, idx_v, rows_v, sem):
        wid = lax.axis_index("s") * _NC + lax.axis_index("c")
        rbase = wid * ROWS_W
        pltpu.sync_copy(pos_hbm.at[pl.ds(rbase, ROWS_W)], idx_v)
        pltpu.sync_copy(x_hbm.at[pl.ds(rbase, ROWS_W)], rows_v)
        pltpu.async_copy(rows_v, xs_hbm.at[idx_v], sem).wait()

    @functools.partial(
        pl.kernel,
        out_type=jax.ShapeDtypeStruct((T, D), jnp.float32),
        mesh=mesh,
        scratch_types=scratch,
    )
    def combine_k(ys_hbm, pos_hbm, out_hbm, idx_v, rows_v, sem):
        wid = lax.axis_index("s") * _NC + lax.axis_index("c")
        rbase = wid * ROWS_W
        pltpu.sync_copy(pos_hbm.at[pl.ds(rbase, ROWS_W)], idx_v)
        pltpu.async_copy(ys_hbm.at[idx_v], rows_v, sem).wait()
        pltpu.sync_copy(rows_v, out_hbm.at[pl.ds(rbase, ROWS_W)])

    return dispatch_k, combine_k


def _mlp_body(te_ref, xs_ref, w1_ref, b1_ref, w2_ref, b2_ref, ys_ref):
    del te_ref
    xt = xs_ref[...]
    h = jnp.dot(xt, w1_ref[0], preferred_element_type=jnp.float32)
    h = h + b1_ref[0]
    h = 0.5 * h * (1.0 + lax.erf(h * 0.7071067811865476))
    o = jnp.dot(h, w2_ref[0], preferred_element_type=jnp.float32)
    ys_ref[...] = o + b2_ref[0]


def _mlp(te, xs, W1, b1, W2, b2):
    grid_spec = pltpu.PrefetchScalarGridSpec(
        num_scalar_prefetch=1,
        grid=(NT,),
        in_specs=[
            pl.BlockSpec((M, D), lambda i, te: (i, 0)),
            pl.BlockSpec((1, D, H), lambda i, te: (te[i], 0, 0)),
            pl.BlockSpec((1, 1, H), lambda i, te: (te[i], 0, 0)),
            pl.BlockSpec((1, H, D), lambda i, te: (te[i], 0, 0)),
            pl.BlockSpec((1, 1, D), lambda i, te: (te[i], 0, 0)),
        ],
        out_specs=pl.BlockSpec((M, D), lambda i, te: (i, 0)),
    )
    return pl.pallas_call(
        _mlp_body,
        grid_spec=grid_spec,
        out_shape=jax.ShapeDtypeStruct((P, D), jnp.float32),
        compiler_params=pltpu.CompilerParams(
            dimension_semantics=("arbitrary",),
        ),
    )(te, xs, W1, b1, W2, b2)


def kernel(x, gate_w, gate_b, W1, b1, W2, b2):
    B, S, _ = x.shape
    x2 = x.reshape(T, D)
    pos2, te2 = _route(x2, gate_w, gate_b.reshape(1, E))
    pos = pos2.reshape(T)
    te = te2.reshape(NTP)[:NT]
    dispatch_k, combine_k = _sc_kernels()
    xs = dispatch_k(x2, pos)
    ys = _mlp(te, xs, W1, b1.reshape(E, 1, H), W2, b2.reshape(E, 1, D))
    out = combine_k(ys, pos)
    return out.reshape(B, S, D), jnp.zeros((), jnp.float32)
